# baseline (speedup 1.0000x reference)
import functools

import jax
import jax.numpy as jnp
from jax import lax
from jax.experimental import pallas as pl
from jax.experimental.pallas import tpu as pltpu
from jax.experimental.pallas import tpu_sc as plsc

F32 = jnp.float32
NC = 2
NS = 16
NW = NC * NS
C = 128


def _dot(a, b):
    return jnp.dot(a, b, preferred_element_type=F32)


def _prep0_body(x_ref, wsr1_ref, wlh1_ref, wlh2_ref, p0s_ref, p0r_ref, p1s_ref):
    x = x_ref[...]
    p0s_ref[...] = _dot(x, wsr1_ref[0:C, :])
    p0r_ref[...] = _dot(x, wsr1_ref[C:2 * C, :])
    wsw2 = _dot(wlh1_ref[0:C, :], wlh2_ref[...])
    p1s_ref[...] = _dot(x, wsw2)


def _prep1_body(x_ref, wlh1_ref, wlh2_ref, p1r_ref):
    wrw2 = _dot(wlh1_ref[C:2 * C, :], wlh2_ref[...])
    p1r_ref[...] = _dot(x_ref[...], wrw2)


def _edge0_body(gs_ref, gr_ref, inv_ref, wsr1_ref, b1_ref, w2_ref, b2_ref,
                w3_ref, b3_ref, out_ref):
    wi = wsr1_ref[2 * C:2 * C + 3, :]
    a = jax.nn.sigmoid(gs_ref[...] + gr_ref[...] + _dot(inv_ref[...], wi)
                       + b1_ref[...])
    bb = jax.nn.sigmoid(_dot(a, w2_ref[...]) + b2_ref[...])
    ew = jax.nn.sigmoid(_dot(bb, w3_ref[...]) + b3_ref[...])
    out_ref[...] = bb * ew


def _edge1_body(gs_ref, gr_ref, inv_ref, wlh1_ref, b1_ref, w2_ref, b2_ref,
                w3_ref, b3_ref, out_ref):
    wi2 = _dot(wlh1_ref[2 * C:2 * C + 6, :], w2_ref[...])
    bias2 = _dot(b1_ref[...], w2_ref[...]) + b2_ref[...]
    pre = gs_ref[...] + gr_ref[...] + _dot(inv_ref[...], wi2) + bias2
    bb = pre * jax.nn.sigmoid(pre)
    ew = jax.nn.sigmoid(_dot(bb, w3_ref[...]) + b3_ref[...])
    out_ref[...] = bb * ew


def _update_body(x_ref, seg_ref, ua_ref, ca_ref, ub_ref, cb_ref, out_ref):
    z = jax.nn.sigmoid(seg_ref[...])
    t = jax.nn.sigmoid(_dot(z, ua_ref[...]) + ca_ref[...])
    out_ref[...] = x_ref[...] + _dot(t, ub_ref[...]) + cb_ref[...]


def _full(shape):
    return pl.BlockSpec(shape, lambda i: (0,) * len(shape))


def _rows(bs, ncols=C):
    return pl.BlockSpec((bs, ncols), lambda i: (i, 0))


def _tc_prep0(x0, wsr1, wlh1, wlh2):
    n = x0.shape[0]
    bs = 1000
    sds = jax.ShapeDtypeStruct((n, C), F32)
    return pl.pallas_call(
        _prep0_body,
        grid=(n // bs,),
        in_specs=[_rows(bs), _full(wsr1.shape), _full(wlh1.shape),
                  _full(wlh2.shape)],
        out_specs=[_rows(bs), _rows(bs), _rows(bs)],
        out_shape=[sds, sds, sds],
    )(x0, wsr1, wlh1, wlh2)


def _tc_prep1(x1, wlh1, wlh2):
    n = x1.shape[0]
    bs = 1000
    return pl.pallas_call(
        _prep1_body,
        grid=(n // bs,),
        in_specs=[_rows(bs), _full(wlh1.shape), _full(wlh2.shape)],
        out_specs=_rows(bs),
        out_shape=jax.ShapeDtypeStruct((n, C), F32),
    )(x1, wlh1, wlh2)


def _tc_edge(body, gs, gr, inv, w1, b1, w2, b2, w3, b3):
    e = gs.shape[0]
    bs = 4000
    return pl.pallas_call(
        body,
        grid=(e // bs,),
        in_specs=[_rows(bs), _rows(bs), _rows(bs, inv.shape[1]),
                  _full(w1.shape), _full(b1.shape), _full(w2.shape),
                  _full(b2.shape), _full(w3.shape), _full(b3.shape)],
        out_specs=_rows(bs),
        out_shape=jax.ShapeDtypeStruct((e, C), F32),
    )(gs, gr, inv, w1, b1, w2, b2, w3, b3)


def _tc_update(x, seg, ua, ca, ub, cb):
    n = x.shape[0]
    bs = 2000 if n % 2000 == 0 else 1000
    return pl.pallas_call(
        _update_body,
        grid=(n // bs,),
        in_specs=[_rows(bs), _rows(bs), _full(ua.shape), _full(ca.shape),
                  _full(ub.shape), _full(cb.shape)],
        out_specs=_rows(bs),
        out_shape=jax.ShapeDtypeStruct((n, C), F32),
    )(x, seg, ua, ca, ub, cb)


_GG = 200


@functools.cache
def _make_gather(n_table, e):
    per_w = e // NW
    n_g = per_w // _GG
    assert per_w % _GG == 0 and n_g % 2 == 0
    mesh = plsc.VectorSubcoreMesh(core_axis_name="c", subcore_axis_name="s",
                                  num_cores=NC, num_subcores=NS)
    @functools.partial(
        pl.kernel, mesh=mesh,
        out_type=jax.ShapeDtypeStruct((e, C), F32),
        scratch_types=[
            pltpu.VMEM((_GG,), jnp.int32),
            pltpu.VMEM((_GG,), jnp.int32),
            pltpu.VMEM((2, _GG, C), F32),
            pltpu.SemaphoreType.DMA,
            pltpu.SemaphoreType.DMA,
            pltpu.SemaphoreType.DMA,
        ],
    )
    def k(table, idx, out, ib0, ib1, rows_v, semi, semg, semw):
        ibs = (ib0, ib1)
        wid = lax.axis_index("s") * NC + lax.axis_index("c")
        base = wid * per_w

        if True:
            def idx_issue(g_, r):
                pltpu.async_copy(idx.at[pl.ds(base + g_ * _GG, _GG)],
                                 ibs[r], semi)

            def idx_drain(r):
                pltpu.make_async_copy(idx.at[pl.ds(0, _GG)], ibs[r],
                                      semi).wait()

            _SPLITS = ((0, 80), (80, 40), (120, 40), (160, 40))

            def rows_issue(r):
                for o_, s_ in _SPLITS:
                    pltpu.async_copy(table.at[ibs[r].at[pl.ds(o_, s_)]],
                                     rows_v.at[r, pl.ds(o_, s_)], semg)

            def rows_drain(r):
                for o_, s_ in _SPLITS:
                    pltpu.make_async_copy(table.at[ibs[r].at[pl.ds(o_, s_)]],
                                          rows_v.at[r, pl.ds(o_, s_)],
                                          semg).wait()

            def wr_issue(g_, r):
                pltpu.async_copy(rows_v.at[r],
                                 out.at[pl.ds(base + g_ * _GG, _GG)], semw)

            def wr_drain(r):
                pltpu.make_async_copy(rows_v.at[r],
                                      out.at[pl.ds(0, _GG)], semw).wait()

            idx_issue(0, 0)
            idx_drain(0)
            rows_issue(0)
            idx_issue(1, 1)

            def pair(p, carry):
                for r in range(2):
                    g = 2 * p + r

                    @pl.when(g >= 1)
                    def _():
                        wr_drain(1 - r)

                    @pl.when(g + 1 < n_g)
                    def _():
                        idx_drain(1 - r)
                        rows_issue(1 - r)

                    rows_drain(r)

                    @pl.when(g + 2 < n_g)
                    def _():
                        idx_issue(g + 2, r)

                    wr_issue(g, r)
                return carry

            lax.fori_loop(0, n_g // 2, pair, 0)
            wr_drain(1)

    return k


_SCH = 80


@functools.cache
def _make_segsum(e, n, bin_rows, pb):
    per_t = e // NS
    nch = per_t // _SCH
    assert per_t % _SCH == 0 and nch % 2 == 0
    acc_rows = -(-(bin_rows + 16) // 256) * 256
    n_dummy = acc_rows - bin_rows
    zrows = acc_rows // NS
    assert acc_rows % NS == 0
    mesh = plsc.VectorSubcoreMesh(core_axis_name="c", subcore_axis_name="s",
                                  num_cores=NC, num_subcores=NS)

    @functools.partial(
        pl.kernel, mesh=mesh,
        out_type=jax.ShapeDtypeStruct((n, C), F32),
        scratch_types=[
            pltpu.VMEM((2, _SCH), jnp.int32),
            pltpu.VMEM((2, _SCH, C), F32),
            pltpu.VMEM((2, _SCH), jnp.int32),
            pltpu.VMEM_SHARED((acc_rows, C), F32),
            pltpu.SemaphoreType.DMA,
            pltpu.SemaphoreType.DMA,
            pltpu.SemaphoreType.DMA,
        ],
    )
    def k(msg, ridx, out, ib, rows_v, idx2, acc, semi, semg, sems):
        c = lax.axis_index("c")
        s = lax.axis_index("s")
        ebase = s * per_t
        dlane = lax.iota(jnp.int32, 16) + s * 16

        def zb(i, carry):
            rows_v[0, i // 8, pl.ds((i % 8) * 16, 16)] = jnp.zeros((16,), F32)
            return carry

        lax.fori_loop(0, _SCH * 8, zb, 0)

        def issue(k_, r):
            off = ebase + k_ * _SCH
            pltpu.async_copy(ridx.at[pl.ds(off, _SCH)], ib.at[r], semi)
            pltpu.async_copy(msg.at[pl.ds(off, _SCH)], rows_v.at[r], semg)

        def drain(r):
            pltpu.make_async_copy(ridx.at[pl.ds(0, _SCH)], ib.at[r],
                                  semi).wait()
            pltpu.make_async_copy(msg.at[pl.ds(0, _SCH)], rows_v.at[r],
                                  semg).wait()

        for b in range(pb):
            lo = (c * pb + b) * bin_rows

            nfull, rem = divmod(zrows, _SCH)
            for i in range(nfull):
                pltpu.sync_copy(rows_v.at[0],
                                acc.at[pl.ds(s * zrows + i * _SCH, _SCH)])
            if rem:
                pltpu.sync_copy(rows_v.at[0, pl.ds(0, rem)],
                                acc.at[pl.ds(s * zrows + nfull * _SCH, rem)])
            plsc.subcore_barrier()

            issue(0, 0)

            def pair(p, carry):
                for r in range(2):
                    kk = 2 * p + r

                    @pl.when(kk >= 2)
                    def _():
                        pltpu.make_async_copy(rows_v.at[1 - r],
                                              acc.at[idx2.at[1 - r]],
                                              sems).wait()

                    @pl.when(kk + 1 < nch)
                    def _():
                        issue(kk + 1, 1 - r)

                    drain(r)
                    for j in range(_SCH // 16):
                        v = ib[r, pl.ds(j * 16, 16)]
                        inb = (v >= lo) & (v < lo + bin_rows)
                        idx2[r, pl.ds(j * 16, 16)] = jnp.where(
                            inb, v - lo,
                            bin_rows + ((dlane + j * 16) % n_dummy))
                    pltpu.async_copy(rows_v.at[r], acc.at[idx2.at[r]], sems,
                                     add=True)
                return carry

            lax.fori_loop(0, nch // 2, pair, 0)
            for r in range(2):
                pltpu.make_async_copy(rows_v.at[r], acc.at[idx2.at[r]],
                                      sems).wait()
            plsc.subcore_barrier()

            for cc in range(NC):
                lo_c = (cc * pb + b) * bin_rows
                cnt = min(bin_rows, n - lo_c)
                if cnt <= 0:
                    continue
                q8 = (cnt // NS) & ~7
                last = cnt - (NS - 1) * q8

                @pl.when(c == cc)
                def _():
                    if q8:
                        @pl.when(s < NS - 1)
                        def _():
                            off = s * q8
                            pltpu.sync_copy(acc.at[pl.ds(off, q8)],
                                            out.at[pl.ds(lo_c + off, q8)])

                    @pl.when(s == NS - 1)
                    def _():
                        off = (NS - 1) * q8
                        pltpu.sync_copy(acc.at[pl.ds(off, last)],
                                        out.at[pl.ds(lo_c + off, last)])

            plsc.subcore_barrier()

            if b + 1 < pb:
                lax.fori_loop(0, _SCH * 8, zb, 0)

    return k


def kernel(x0, x1, adj0_index, inv0, inc1_send, inc1_recv, inv1,
           W_sr1, b_sr1, W_sr2, b_sr2, W_sr3, b_sr3,
           W_lh1, b_lh1, W_lh2, b_lh2, W_lh3, b_lh3,
           U0a, c0a, U0b, c0b, U1a, c1a, U1b, c1b):
    n0 = x0.shape[0]
    n1 = x1.shape[0]
    e0 = adj0_index.shape[1]
    e1 = inc1_send.shape[0]

    send0 = adj0_index[0].astype(jnp.int32)
    recv0 = adj0_index[1].astype(jnp.int32)
    send1 = inc1_send.astype(jnp.int32)
    recv1 = inc1_recv.astype(jnp.int32)

    r = lambda v: v.reshape(1, -1)

    p0s, p0r, p1s = _tc_prep0(x0, W_sr1, W_lh1, W_lh2)
    p1r = _tc_prep1(x1, W_lh1, W_lh2)

    g0s = _make_gather(n0, e0)(p0s, send0)
    g0r = _make_gather(n0, e0)(p0r, recv0)
    g1s = _make_gather(n0, e1)(p1s, send1)
    g1r = _make_gather(n1, e1)(p1r, recv1)

    msg0 = _tc_edge(_edge0_body, g0s, g0r, inv0, W_sr1, r(b_sr1),
                    W_sr2, r(b_sr2), W_sr3, r(b_sr3))
    msg1 = _tc_edge(_edge1_body, g1s, g1r, inv1, W_lh1, r(b_lh1),
                    W_lh2, r(b_lh2), W_lh3, r(b_lh3))

    seg0 = _make_segsum(e0, n0, 5000, 1)(msg0, recv0)
    seg1 = _make_segsum(e1, n1, 13440, 6)(msg1, recv1)

    out0 = _tc_update(x0, seg0, U0a, r(c0a), U0b, r(c0b))
    out1 = _tc_update(x1, seg1, U1a, r(c1a), U1b, r(c1b))
    return (out0, out1)


# speedup vs baseline: 1.0182x; 1.0182x over previous
import functools

import jax
import jax.numpy as jnp
from jax import lax
from jax.experimental import pallas as pl
from jax.experimental.pallas import tpu as pltpu
from jax.experimental.pallas import tpu_sc as plsc

F32 = jnp.float32
NC = 2
NS = 16
NW = NC * NS
C = 128


def _dot(a, b):
    return jnp.dot(a, b, preferred_element_type=F32)


def _prep0_body(x_ref, wsr1_ref, wlh1_ref, wlh2_ref, p0s_ref, p0r_ref, p1s_ref):
    x = x_ref[...]
    p0s_ref[...] = _dot(x, wsr1_ref[0:C, :])
    p0r_ref[...] = _dot(x, wsr1_ref[C:2 * C, :])
    wsw2 = _dot(wlh1_ref[0:C, :], wlh2_ref[...])
    p1s_ref[...] = _dot(x, wsw2)


def _prep1_body(x_ref, wlh1_ref, wlh2_ref, p1r_ref):
    wrw2 = _dot(wlh1_ref[C:2 * C, :], wlh2_ref[...])
    p1r_ref[...] = _dot(x_ref[...], wrw2)


def _edge0_body(gs_ref, gr_ref, inv_ref, wsr1_ref, b1_ref, w2_ref, b2_ref,
                w3_ref, b3_ref, out_ref):
    wi = wsr1_ref[2 * C:2 * C + 3, :]
    a = jax.nn.sigmoid(gs_ref[...] + gr_ref[...] + _dot(inv_ref[...], wi)
                       + b1_ref[...])
    bb = jax.nn.sigmoid(_dot(a, w2_ref[...]) + b2_ref[...])
    ew = jax.nn.sigmoid(_dot(bb, w3_ref[...]) + b3_ref[...])
    out_ref[...] = bb * ew


def _edge1_body(gs_ref, gr_ref, inv_ref, wlh1_ref, b1_ref, w2_ref, b2_ref,
                w3_ref, b3_ref, out_ref):
    wi2 = _dot(wlh1_ref[2 * C:2 * C + 6, :], w2_ref[...])
    bias2 = _dot(b1_ref[...], w2_ref[...]) + b2_ref[...]
    pre = gs_ref[...] + gr_ref[...] + _dot(inv_ref[...], wi2) + bias2
    bb = pre * jax.nn.sigmoid(pre)
    ew = jax.nn.sigmoid(_dot(bb, w3_ref[...]) + b3_ref[...])
    out_ref[...] = bb * ew


def _update_body(x_ref, seg_ref, ua_ref, ca_ref, ub_ref, cb_ref, out_ref):
    z = jax.nn.sigmoid(seg_ref[...])
    t = jax.nn.sigmoid(_dot(z, ua_ref[...]) + ca_ref[...])
    out_ref[...] = x_ref[...] + _dot(t, ub_ref[...]) + cb_ref[...]


def _full(shape):
    return pl.BlockSpec(shape, lambda i: (0,) * len(shape))


def _rows(bs, ncols=C):
    return pl.BlockSpec((bs, ncols), lambda i: (i, 0))


def _tc_prep0(x0, wsr1, wlh1, wlh2):
    n = x0.shape[0]
    bs = 1000
    sds = jax.ShapeDtypeStruct((n, C), F32)
    return pl.pallas_call(
        _prep0_body,
        grid=(n // bs,),
        in_specs=[_rows(bs), _full(wsr1.shape), _full(wlh1.shape),
                  _full(wlh2.shape)],
        out_specs=[_rows(bs), _rows(bs), _rows(bs)],
        out_shape=[sds, sds, sds],
    )(x0, wsr1, wlh1, wlh2)


def _tc_prep1(x1, wlh1, wlh2):
    n = x1.shape[0]
    bs = 1000
    return pl.pallas_call(
        _prep1_body,
        grid=(n // bs,),
        in_specs=[_rows(bs), _full(wlh1.shape), _full(wlh2.shape)],
        out_specs=_rows(bs),
        out_shape=jax.ShapeDtypeStruct((n, C), F32),
    )(x1, wlh1, wlh2)


def _tc_edge(body, gs, gr, inv, w1, b1, w2, b2, w3, b3):
    e = gs.shape[0]
    bs = 4000
    return pl.pallas_call(
        body,
        grid=(e // bs,),
        in_specs=[_rows(bs), _rows(bs), _rows(bs, inv.shape[1]),
                  _full(w1.shape), _full(b1.shape), _full(w2.shape),
                  _full(b2.shape), _full(w3.shape), _full(b3.shape)],
        out_specs=_rows(bs),
        out_shape=jax.ShapeDtypeStruct((e, C), F32),
    )(gs, gr, inv, w1, b1, w2, b2, w3, b3)


def _tc_update(x, seg, ua, ca, ub, cb):
    n = x.shape[0]
    bs = 2000 if n % 2000 == 0 else 1000
    return pl.pallas_call(
        _update_body,
        grid=(n // bs,),
        in_specs=[_rows(bs), _rows(bs), _full(ua.shape), _full(ca.shape),
                  _full(ub.shape), _full(cb.shape)],
        out_specs=_rows(bs),
        out_shape=jax.ShapeDtypeStruct((n, C), F32),
    )(x, seg, ua, ca, ub, cb)


def _update2_body(x_ref, sa_ref, sb_ref, ua_ref, ca_ref, ub_ref, cb_ref,
                  out_ref):
    z = jax.nn.sigmoid(sa_ref[0] + sb_ref[0])
    t = jax.nn.sigmoid(_dot(z, ua_ref[...]) + ca_ref[...])
    out_ref[...] = x_ref[...] + _dot(t, ub_ref[...]) + cb_ref[...]


def _tc_update2(x, seg2, ua, ca, ub, cb):
    n = x.shape[0]
    bs = 2000 if n % 2000 == 0 else 1000
    spec_a = pl.BlockSpec((1, bs, C), lambda i: (0, i, 0))
    spec_b = pl.BlockSpec((1, bs, C), lambda i: (1, i, 0))
    return pl.pallas_call(
        _update2_body,
        grid=(n // bs,),
        in_specs=[_rows(bs), spec_a, spec_b, _full(ua.shape), _full(ca.shape),
                  _full(ub.shape), _full(cb.shape)],
        out_specs=_rows(bs),
        out_shape=jax.ShapeDtypeStruct((n, C), F32),
    )(x, seg2, seg2, ua, ca, ub, cb)


_GG = 200


@functools.cache
def _make_gather(n_table, e):
    per_w = e // NW
    n_g = per_w // _GG
    assert per_w % _GG == 0 and n_g % 2 == 0
    mesh = plsc.VectorSubcoreMesh(core_axis_name="c", subcore_axis_name="s",
                                  num_cores=NC, num_subcores=NS)
    @functools.partial(
        pl.kernel, mesh=mesh,
        out_type=jax.ShapeDtypeStruct((e, C), F32),
        scratch_types=[
            pltpu.VMEM((_GG,), jnp.int32),
            pltpu.VMEM((_GG,), jnp.int32),
            pltpu.VMEM((2, _GG, C), F32),
            pltpu.SemaphoreType.DMA,
            pltpu.SemaphoreType.DMA,
            pltpu.SemaphoreType.DMA,
        ],
    )
    def k(table, idx, out, ib0, ib1, rows_v, semi, semg, semw):
        ibs = (ib0, ib1)
        wid = lax.axis_index("s") * NC + lax.axis_index("c")
        base = wid * per_w

        if True:
            def idx_issue(g_, r):
                pltpu.async_copy(idx.at[pl.ds(base + g_ * _GG, _GG)],
                                 ibs[r], semi)

            def idx_drain(r):
                pltpu.make_async_copy(idx.at[pl.ds(0, _GG)], ibs[r],
                                      semi).wait()

            _SPLITS = ((0, 80), (80, 40), (120, 40), (160, 40))

            def rows_issue(r):
                for o_, s_ in _SPLITS:
                    pltpu.async_copy(table.at[ibs[r].at[pl.ds(o_, s_)]],
                                     rows_v.at[r, pl.ds(o_, s_)], semg)

            def rows_drain(r):
                for o_, s_ in _SPLITS:
                    pltpu.make_async_copy(table.at[ibs[r].at[pl.ds(o_, s_)]],
                                          rows_v.at[r, pl.ds(o_, s_)],
                                          semg).wait()

            def wr_issue(g_, r):
                pltpu.async_copy(rows_v.at[r],
                                 out.at[pl.ds(base + g_ * _GG, _GG)], semw)

            def wr_drain(r):
                pltpu.make_async_copy(rows_v.at[r],
                                      out.at[pl.ds(0, _GG)], semw).wait()

            idx_issue(0, 0)
            idx_drain(0)
            rows_issue(0)
            idx_issue(1, 1)

            def pair(p, carry):
                for r in range(2):
                    g = 2 * p + r

                    @pl.when(g >= 1)
                    def _():
                        wr_drain(1 - r)

                    @pl.when(g + 1 < n_g)
                    def _():
                        idx_drain(1 - r)
                        rows_issue(1 - r)

                    rows_drain(r)

                    @pl.when(g + 2 < n_g)
                    def _():
                        idx_issue(g + 2, r)

                    wr_issue(g, r)
                return carry

            lax.fori_loop(0, n_g // 2, pair, 0)
            wr_drain(1)

    return k


_SCH = 80


@functools.cache
def _make_segsum(e, n, bin_rows, pb):
    per_t = e // NS
    nch = per_t // _SCH
    assert per_t % _SCH == 0 and nch % 2 == 0
    acc_rows = -(-(bin_rows + 16) // 256) * 256
    n_dummy = acc_rows - bin_rows
    zrows = acc_rows // NS
    assert acc_rows % NS == 0
    mesh = plsc.VectorSubcoreMesh(core_axis_name="c", subcore_axis_name="s",
                                  num_cores=NC, num_subcores=NS)

    @functools.partial(
        pl.kernel, mesh=mesh,
        out_type=jax.ShapeDtypeStruct((n, C), F32),
        scratch_types=[
            pltpu.VMEM((2, _SCH), jnp.int32),
            pltpu.VMEM((2, _SCH, C), F32),
            pltpu.VMEM((2, _SCH), jnp.int32),
            pltpu.VMEM_SHARED((acc_rows, C), F32),
            pltpu.SemaphoreType.DMA,
            pltpu.SemaphoreType.DMA,
            pltpu.SemaphoreType.DMA,
        ],
    )
    def k(msg, ridx, out, ib, rows_v, idx2, acc, semi, semg, sems):
        c = lax.axis_index("c")
        s = lax.axis_index("s")
        ebase = s * per_t
        dlane = lax.iota(jnp.int32, 16) + s * 16

        def zb(i, carry):
            rows_v[0, i // 8, pl.ds((i % 8) * 16, 16)] = jnp.zeros((16,), F32)
            return carry

        lax.fori_loop(0, _SCH * 8, zb, 0)

        def issue(k_, r):
            off = ebase + k_ * _SCH
            pltpu.async_copy(ridx.at[pl.ds(off, _SCH)], ib.at[r], semi)
            pltpu.async_copy(msg.at[pl.ds(off, _SCH)], rows_v.at[r], semg)

        def drain(r):
            pltpu.make_async_copy(ridx.at[pl.ds(0, _SCH)], ib.at[r],
                                  semi).wait()
            pltpu.make_async_copy(msg.at[pl.ds(0, _SCH)], rows_v.at[r],
                                  semg).wait()

        for b in range(pb):
            lo = (c * pb + b) * bin_rows

            nfull, rem = divmod(zrows, _SCH)
            for i in range(nfull):
                pltpu.sync_copy(rows_v.at[0],
                                acc.at[pl.ds(s * zrows + i * _SCH, _SCH)])
            if rem:
                pltpu.sync_copy(rows_v.at[0, pl.ds(0, rem)],
                                acc.at[pl.ds(s * zrows + nfull * _SCH, rem)])
            plsc.subcore_barrier()

            issue(0, 0)

            def pair(p, carry):
                for r in range(2):
                    kk = 2 * p + r

                    @pl.when(kk >= 2)
                    def _():
                        pltpu.make_async_copy(rows_v.at[1 - r],
                                              acc.at[idx2.at[1 - r]],
                                              sems).wait()

                    @pl.when(kk + 1 < nch)
                    def _():
                        issue(kk + 1, 1 - r)

                    drain(r)
                    for j in range(_SCH // 16):
                        v = ib[r, pl.ds(j * 16, 16)]
                        inb = (v >= lo) & (v < lo + bin_rows)
                        idx2[r, pl.ds(j * 16, 16)] = jnp.where(
                            inb, v - lo,
                            bin_rows + ((dlane + j * 16) % n_dummy))
                    pltpu.async_copy(rows_v.at[r], acc.at[idx2.at[r]], sems,
                                     add=True)
                return carry

            lax.fori_loop(0, nch // 2, pair, 0)
            for r in range(2):
                pltpu.make_async_copy(rows_v.at[r], acc.at[idx2.at[r]],
                                      sems).wait()
            plsc.subcore_barrier()

            for cc in range(NC):
                lo_c = (cc * pb + b) * bin_rows
                cnt = min(bin_rows, n - lo_c)
                if cnt <= 0:
                    continue
                q8 = (cnt // NS) & ~7
                last = cnt - (NS - 1) * q8

                @pl.when(c == cc)
                def _():
                    if q8:
                        @pl.when(s < NS - 1)
                        def _():
                            off = s * q8
                            pltpu.sync_copy(acc.at[pl.ds(off, q8)],
                                            out.at[pl.ds(lo_c + off, q8)])

                    @pl.when(s == NS - 1)
                    def _():
                        off = (NS - 1) * q8
                        pltpu.sync_copy(acc.at[pl.ds(off, last)],
                                        out.at[pl.ds(lo_c + off, last)])

            plsc.subcore_barrier()

            if b + 1 < pb:
                lax.fori_loop(0, _SCH * 8, zb, 0)

    return k


_SCHF = 40


@functools.cache
def _make_segsum_full(e, n):
    per_t = e // NW
    nch = per_t // _SCHF
    assert per_t % _SCHF == 0 and nch % 2 == 0
    acc_rows = -(-(n + 16) // 256) * 256
    zrows = acc_rows // NS
    assert acc_rows % NS == 0
    mesh = plsc.VectorSubcoreMesh(core_axis_name="c", subcore_axis_name="s",
                                  num_cores=NC, num_subcores=NS)

    @functools.partial(
        pl.kernel, mesh=mesh,
        out_type=jax.ShapeDtypeStruct((NC, n, C), F32),
        scratch_types=[
            pltpu.VMEM((2, _SCHF), jnp.int32),
            pltpu.VMEM((2, _SCHF, C), F32),
            pltpu.VMEM((2, _SCHF), jnp.int32),
            pltpu.VMEM_SHARED((acc_rows, C), F32),
            pltpu.SemaphoreType.DMA,
            pltpu.SemaphoreType.DMA,
            pltpu.SemaphoreType.DMA,
        ],
    )
    def k(msg, ridx, out, ib, rows_v, idx2, acc, semi, semg, sems):
        c = lax.axis_index("c")
        s = lax.axis_index("s")
        ebase = c * (e // NC) + s * per_t

        def zb(i, carry):
            rows_v[0, i // 8, pl.ds((i % 8) * 16, 16)] = jnp.zeros((16,), F32)
            return carry

        lax.fori_loop(0, _SCHF * 8, zb, 0)

        nfull, rem = divmod(zrows, _SCHF)
        for i in range(nfull):
            pltpu.sync_copy(rows_v.at[0],
                            acc.at[pl.ds(s * zrows + i * _SCHF, _SCHF)])
        if rem:
            pltpu.sync_copy(rows_v.at[0, pl.ds(0, rem)],
                            acc.at[pl.ds(s * zrows + nfull * _SCHF, rem)])
        plsc.subcore_barrier()

        def issue(k_, r):
            off = ebase + k_ * _SCHF
            pltpu.async_copy(ridx.at[pl.ds(off, _SCHF)], ib.at[r], semi)
            pltpu.async_copy(msg.at[pl.ds(off, _SCHF)], rows_v.at[r], semg)

        def drain(r):
            pltpu.make_async_copy(ridx.at[pl.ds(0, _SCHF)], ib.at[r],
                                  semi).wait()
            pltpu.make_async_copy(msg.at[pl.ds(0, _SCHF)], rows_v.at[r],
                                  semg).wait()

        issue(0, 0)

        def pair(p, carry):
            for r in range(2):
                kk = 2 * p + r

                @pl.when(kk >= 2)
                def _():
                    pltpu.make_async_copy(rows_v.at[1 - r],
                                          acc.at[idx2.at[1 - r]],
                                          sems).wait()

                @pl.when(kk + 1 < nch)
                def _():
                    issue(kk + 1, 1 - r)

                drain(r)
                for j in range(_SCHF // 16):
                    idx2[r, pl.ds(j * 16, 16)] = ib[r, pl.ds(j * 16, 16)]
                idx2[r, pl.ds(_SCHF - 16, 16)] = ib[r, pl.ds(_SCHF - 16, 16)]
                pltpu.async_copy(rows_v.at[r], acc.at[idx2.at[r]], sems,
                                 add=True)
            return carry

        lax.fori_loop(0, nch // 2, pair, 0)
        for r in range(2):
            pltpu.make_async_copy(rows_v.at[r], acc.at[idx2.at[r]],
                                  sems).wait()
        plsc.subcore_barrier()

        q8 = (n // NS) & ~7
        last = n - (NS - 1) * q8
        for cc in range(NC):
            @pl.when(c == cc)
            def _():
                if q8:
                    @pl.when(s < NS - 1)
                    def _():
                        off = s * q8
                        pltpu.sync_copy(acc.at[pl.ds(off, q8)],
                                        out.at[cc, pl.ds(off, q8)])

                @pl.when(s == NS - 1)
                def _():
                    off = (NS - 1) * q8
                    pltpu.sync_copy(acc.at[pl.ds(off, last)],
                                    out.at[cc, pl.ds(off, last)])

    return k


def kernel(x0, x1, adj0_index, inv0, inc1_send, inc1_recv, inv1,
           W_sr1, b_sr1, W_sr2, b_sr2, W_sr3, b_sr3,
           W_lh1, b_lh1, W_lh2, b_lh2, W_lh3, b_lh3,
           U0a, c0a, U0b, c0b, U1a, c1a, U1b, c1b):
    n0 = x0.shape[0]
    n1 = x1.shape[0]
    e0 = adj0_index.shape[1]
    e1 = inc1_send.shape[0]

    send0 = adj0_index[0].astype(jnp.int32)
    recv0 = adj0_index[1].astype(jnp.int32)
    send1 = inc1_send.astype(jnp.int32)
    recv1 = inc1_recv.astype(jnp.int32)

    r = lambda v: v.reshape(1, -1)

    p0s, p0r, p1s = _tc_prep0(x0, W_sr1, W_lh1, W_lh2)
    p1r = _tc_prep1(x1, W_lh1, W_lh2)

    g0s = _make_gather(n0, e0)(p0s, send0)
    g0r = _make_gather(n0, e0)(p0r, recv0)
    g1s = _make_gather(n0, e1)(p1s, send1)
    g1r = _make_gather(n1, e1)(p1r, recv1)

    msg0 = _tc_edge(_edge0_body, g0s, g0r, inv0, W_sr1, r(b_sr1),
                    W_sr2, r(b_sr2), W_sr3, r(b_sr3))
    msg1 = _tc_edge(_edge1_body, g1s, g1r, inv1, W_lh1, r(b_lh1),
                    W_lh2, r(b_lh2), W_lh3, r(b_lh3))

    seg0 = _make_segsum_full(e0, n0)(msg0, recv0)
    seg1 = _make_segsum(e1, n1, 13440, 6)(msg1, recv1)

    out0 = _tc_update2(x0, seg0, U0a, r(c0a), U0b, r(c0b))
    out1 = _tc_update(x1, seg1, U1a, r(c1a), U1b, r(c1b))
    return (out0, out1)
